# baseline (speedup 1.0000x reference)
import functools

import jax
import jax.numpy as jnp
from jax import lax
from jax.experimental import pallas as pl
from jax.experimental.pallas import tpu as pltpu
from jax.experimental.pallas import tpu_sc as plsc

N = 10000
E = 320000
D = 128
G = 256

NC = 2
NS = 16
NW = NC * NS

N_PAD = 10240
PAD_DST = 10200
E_PAD = 327680
EROWS = E_PAD // 128
EROWS_PER_W = EROWS // NW
ROWS_PER_TILE = N_PAD // NS
CW = 128


CROWS = 16
EROWS_C = (2304, 256)
ERPT_C = (EROWS_C[0] // NS, EROWS_C[1] // NS)


def _sc_agg(x_hbm, src_hbm, dst_hbm, z2d_hbm, out_hbm,
            idx_s, idx_d, rows_a, rows_b, sem_a, sem_b, sem_sa, sem_sb, acc):
    cid = lax.axis_index("c")
    sid = lax.axis_index("s")

    base = sid * ROWS_PER_TILE
    pltpu.sync_copy(z2d_hbm.at[pl.ds(base, ROWS_PER_TILE)],
                    acc.at[pl.ds(base, ROWS_PER_TILE)])
    plsc.subcore_barrier()

    tile_row0 = jnp.where(cid == 0, sid * ERPT_C[0],
                          EROWS_C[0] + sid * ERPT_C[1])
    nchunk = jnp.where(cid == 0, ERPT_C[0] // CROWS, ERPT_C[1] // CROWS)

    def _gather(j2, dst_half, sem):
        pltpu.async_copy(x_hbm.at[idx_s.at[j2]], dst_half, sem)

    def _fire(j, rows, sem):
        _gather(2 * j, rows.at[pl.ds(0, 64)], sem)
        _gather(2 * j + 1, rows.at[pl.ds(64, 64)], sem)

    def _wait(j, rows, sem):
        pltpu.make_async_copy(x_hbm.at[idx_s.at[2 * j]],
                              rows.at[pl.ds(0, 64)], sem).wait()
        pltpu.make_async_copy(x_hbm.at[idx_s.at[2 * j + 1]],
                              rows.at[pl.ds(64, 64)], sem).wait()

    @pl.loop(0, nchunk)
    def _(c):
        erow0 = pl.multiple_of(tile_row0 + c * CROWS, 8)
        pltpu.sync_copy(src_hbm.at[pl.ds(2 * erow0, 2 * CROWS)],
                        idx_s.at[pl.ds(0, 2 * CROWS)])
        pltpu.sync_copy(src_hbm.at[pl.ds(0, 4)],
                        idx_s.at[pl.ds(2 * CROWS, 4)])
        pltpu.sync_copy(dst_hbm.at[pl.ds(erow0, CROWS)], idx_d)

        _fire(0, rows_a, sem_a)
        _fire(1, rows_b, sem_b)

        @pl.loop(0, CROWS // 2)
        def _(jj):
            j = 2 * jj
            _wait(j, rows_a, sem_a)
            pltpu.async_copy(rows_a, acc.at[idx_d.at[j]], sem_sa, add=True)
            _wait(j + 1, rows_b, sem_b)
            pltpu.async_copy(rows_b, acc.at[idx_d.at[j + 1]], sem_sb, add=True)
            pltpu.make_async_copy(rows_a, acc.at[idx_d.at[j]], sem_sa).wait()
            _fire(j + 2, rows_a, sem_a)
            pltpu.make_async_copy(rows_b, acc.at[idx_d.at[j + 1]],
                                  sem_sb).wait()
            _fire(j + 3, rows_b, sem_b)

        _wait(CROWS, rows_a, sem_a)
        _wait(CROWS + 1, rows_b, sem_b)

    plsc.subcore_barrier()

    pltpu.sync_copy(acc.at[pl.ds(base, ROWS_PER_TILE)],
                    out_hbm.at[cid, pl.ds(base, ROWS_PER_TILE)])


def _sc_cnt(dst_hbm, zc_hbm, ones_hbm, cnt_hbm, idx_d, ones_v, cnt, sem):
    cid = lax.axis_index("c")
    sid = lax.axis_index("s")
    wid = sid * NC + cid

    base = sid * ROWS_PER_TILE
    pltpu.sync_copy(zc_hbm.at[pl.ds(base, ROWS_PER_TILE)],
                    cnt.at[pl.ds(base, ROWS_PER_TILE)])
    pltpu.sync_copy(ones_hbm, ones_v)
    plsc.subcore_barrier()

    erow0 = wid * EROWS_PER_W
    pltpu.sync_copy(dst_hbm.at[pl.ds(erow0, EROWS_PER_W)], idx_d)

    @pl.loop(0, EROWS_PER_W)
    def _(j):
        pltpu.sync_copy(ones_v, cnt.at[idx_d.at[j]], add=True)

    plsc.subcore_barrier()
    pltpu.sync_copy(cnt.at[pl.ds(base, ROWS_PER_TILE)],
                    cnt_hbm.at[cid, pl.ds(base, ROWS_PER_TILE)])


@functools.lru_cache(maxsize=None)
def _sc_agg_call():
    mesh = plsc.VectorSubcoreMesh(core_axis_name="c", subcore_axis_name="s",
                                  num_cores=NC, num_subcores=NS)
    return pl.kernel(
        _sc_agg,
        out_type=[jax.ShapeDtypeStruct((NC, N_PAD, D), jnp.float32)],
        mesh=mesh,
        scratch_types=[
            pltpu.VMEM((2 * (CROWS + 2), 64), jnp.int32),
            pltpu.VMEM((CROWS, 128), jnp.int32),
            pltpu.VMEM((128, D), jnp.float32),
            pltpu.VMEM((128, D), jnp.float32),
            pltpu.SemaphoreType.DMA,
            pltpu.SemaphoreType.DMA,
            pltpu.SemaphoreType.DMA,
            pltpu.SemaphoreType.DMA,
            pltpu.VMEM_SHARED((N_PAD, D), jnp.float32),
        ],
        name="sc_seg_agg")


@functools.lru_cache(maxsize=None)
def _sc_cnt_call():
    mesh = plsc.VectorSubcoreMesh(core_axis_name="c", subcore_axis_name="s",
                                  num_cores=NC, num_subcores=NS)
    return pl.kernel(
        _sc_cnt,
        out_type=[jax.ShapeDtypeStruct((NC, N_PAD, CW), jnp.float32)],
        mesh=mesh,
        scratch_types=[
            pltpu.VMEM((EROWS_PER_W, 128), jnp.int32),
            pltpu.VMEM((128, CW), jnp.float32),
            pltpu.VMEM_SHARED((N_PAD, CW), jnp.float32),
            pltpu.SemaphoreType.DMA,
        ],
        name="sc_seg_cnt")


BLK = 2000
NBLK = N // BLK


def _tc_layer(parts_ref, cnt_ref, x_ref, wl_ref, wr_ref, b_ref, o_ref):
    s = parts_ref[0] + parts_ref[1]
    deg = cnt_ref[0, :, 0:1] + cnt_ref[1, :, 0:1]
    mean = s / jnp.maximum(deg, 1.0)
    h = jnp.dot(mean, wl_ref[...], preferred_element_type=jnp.float32)
    h = h + jnp.dot(x_ref[...], wr_ref[...], preferred_element_type=jnp.float32)
    o_ref[...] = jnp.maximum(h + b_ref[...], 0.0)


def _tc_layer_call(parts, cnt, x, wl, wr, b):
    return pl.pallas_call(
        _tc_layer,
        grid=(NBLK,),
        in_specs=[
            pl.BlockSpec((NC, BLK, D), lambda i: (0, i, 0)),
            pl.BlockSpec((NC, BLK, CW), lambda i: (0, i, 0)),
            pl.BlockSpec((BLK, D), lambda i: (i, 0)),
            pl.BlockSpec((D, D), lambda i: (0, 0)),
            pl.BlockSpec((D, D), lambda i: (0, 0)),
            pl.BlockSpec((1, D), lambda i: (0, 0)),
        ],
        out_specs=pl.BlockSpec((BLK, D), lambda i: (i, 0)),
        out_shape=jax.ShapeDtypeStruct((N, D), jnp.float32),
    )(parts, cnt, x, wl, wr, b)


def _tc_final(parts_ref, cnt_ref, h1_ref, wl_ref, wr_ref, b_ref,
              batch_ref, w3_ref, b3_ref, w4_ref, b4_ref, o_ref,
              pool_acc, cntg_acc):
    i = pl.program_id(0)
    s = parts_ref[0] + parts_ref[1]
    deg = cnt_ref[0, :, 0:1] + cnt_ref[1, :, 0:1]
    mean = s / jnp.maximum(deg, 1.0)
    h = jnp.dot(mean, wl_ref[...], preferred_element_type=jnp.float32)
    h = h + jnp.dot(h1_ref[...], wr_ref[...], preferred_element_type=jnp.float32)
    h2 = jnp.maximum(h + b_ref[...], 0.0)

    giota = lax.broadcasted_iota(jnp.int32, (G, BLK), 0)
    onehot_t = (giota == batch_ref[0]).astype(jnp.float32)

    @pl.when(i == 0)
    def _():
        pool_acc[...] = jnp.zeros_like(pool_acc)
        cntg_acc[...] = jnp.zeros_like(cntg_acc)

    pool_acc[...] += jnp.dot(onehot_t, h2, preferred_element_type=jnp.float32)
    cntg_acc[...] += jnp.sum(onehot_t, axis=1, keepdims=True)

    @pl.when(i == NBLK - 1)
    def _():
        pooled = pool_acc[...] / jnp.maximum(cntg_acc[...], 1.0)
        z = jnp.dot(pooled, w3_ref[...], preferred_element_type=jnp.float32)
        z = jnp.maximum(z + b3_ref[...], 0.0)
        o_ref[...] = jnp.dot(z, w4_ref[...],
                             preferred_element_type=jnp.float32) + b4_ref[...]


def _tc_final_call(parts, cnt, h1, wl, wr, b, batch3, w3, b3, w4, b4):
    return pl.pallas_call(
        _tc_final,
        grid=(NBLK,),
        in_specs=[
            pl.BlockSpec((NC, BLK, D), lambda i: (0, i, 0)),
            pl.BlockSpec((NC, BLK, CW), lambda i: (0, i, 0)),
            pl.BlockSpec((BLK, D), lambda i: (i, 0)),
            pl.BlockSpec((D, D), lambda i: (0, 0)),
            pl.BlockSpec((D, D), lambda i: (0, 0)),
            pl.BlockSpec((1, D), lambda i: (0, 0)),
            pl.BlockSpec((1, 1, BLK), lambda i: (i, 0, 0)),
            pl.BlockSpec((D, D), lambda i: (0, 0)),
            pl.BlockSpec((1, D), lambda i: (0, 0)),
            pl.BlockSpec((D, D), lambda i: (0, 0)),
            pl.BlockSpec((1, D), lambda i: (0, 0)),
        ],
        out_specs=pl.BlockSpec((G, D), lambda i: (0, 0)),
        out_shape=jax.ShapeDtypeStruct((G, D), jnp.float32),
        scratch_shapes=[
            pltpu.VMEM((G, D), jnp.float32),
            pltpu.VMEM((G, 1), jnp.float32),
        ],
    )(parts, cnt, h1, wl, wr, b, batch3, w3, b3, w4, b4)


def kernel(x, edge_index, batch, W1l, b1l, W1r, W2l, b2l, W2r, W3, b3, W4, b4):
    src = edge_index[0]
    dst = edge_index[1]
    npad = E_PAD - E
    src2 = jnp.concatenate(
        [src, jnp.zeros((npad,), jnp.int32)]).reshape(2 * EROWS, 64)
    dst2 = jnp.concatenate(
        [dst, jnp.full((npad,), PAD_DST, jnp.int32)]).reshape(EROWS, 128)
    z2d = jnp.zeros((N_PAD, D), jnp.float32)
    ones = jnp.ones((128, CW), jnp.float32)

    (cnt,) = _sc_cnt_call()(dst2, z2d, ones)
    (parts1,) = _sc_agg_call()(x, src2, dst2, z2d)
    h1 = _tc_layer_call(parts1, cnt, x, W1l, W1r, b1l.reshape(1, D))
    (parts2,) = _sc_agg_call()(h1, src2, dst2, z2d)
    batch3 = batch.reshape(NBLK, 1, BLK)
    return _tc_final_call(parts2, cnt, h1, W2l, W2r, b2l.reshape(1, D),
                          batch3, W3, b3.reshape(1, D), W4, b4.reshape(1, D))


# speedup vs baseline: 1.0178x; 1.0178x over previous
import functools

import jax
import jax.numpy as jnp
from jax import lax
from jax.experimental import pallas as pl
from jax.experimental.pallas import tpu as pltpu
from jax.experimental.pallas import tpu_sc as plsc

N = 10000
E = 320000
D = 128
G = 256

NC = 2
NS = 16
NW = NC * NS

N_PAD = 10240
PAD_DST = 10200
E_PAD = 327680
EROWS = E_PAD // 128
EROWS_PER_W = EROWS // NW
ROWS_PER_TILE = N_PAD // NS
CW = 128


CROWS = 16
EROWS_C = (2304, 256)
ERPT_C = (EROWS_C[0] // NS, EROWS_C[1] // NS)


def _sc_agg(x_hbm, src_hbm, dst_hbm, z2d_hbm, out_hbm,
            idx_s, idx_d, rows_a, rows_b, sem_a, sem_b, sem_sa, sem_sb, acc):
    cid = lax.axis_index("c")
    sid = lax.axis_index("s")

    base = sid * ROWS_PER_TILE
    pltpu.sync_copy(z2d_hbm.at[pl.ds(base, ROWS_PER_TILE)],
                    acc.at[pl.ds(base, ROWS_PER_TILE)])
    plsc.subcore_barrier()

    tile_row0 = jnp.where(cid == 0, sid * ERPT_C[0],
                          EROWS_C[0] + sid * ERPT_C[1])
    nchunk = jnp.where(cid == 0, ERPT_C[0] // CROWS, ERPT_C[1] // CROWS)

    @pl.loop(0, nchunk)
    def _(c):
        erow0 = pl.multiple_of(tile_row0 + c * CROWS, 8)
        pltpu.sync_copy(src_hbm.at[pl.ds(erow0, CROWS)],
                        idx_s.at[pl.ds(0, CROWS)])
        pltpu.sync_copy(src_hbm.at[pl.ds(0, 2)],
                        idx_s.at[pl.ds(CROWS, 2)])
        pltpu.sync_copy(dst_hbm.at[pl.ds(erow0, CROWS)], idx_d)

        pltpu.async_copy(x_hbm.at[idx_s.at[0]], rows_a, sem_a)
        pltpu.async_copy(x_hbm.at[idx_s.at[1]], rows_b, sem_b)

        @pl.loop(0, CROWS // 2)
        def _(jj):
            j = 2 * jj
            pltpu.make_async_copy(x_hbm.at[idx_s.at[j]], rows_a, sem_a).wait()
            pltpu.async_copy(rows_a, acc.at[idx_d.at[j]], sem_sa, add=True)
            pltpu.make_async_copy(x_hbm.at[idx_s.at[j + 1]], rows_b,
                                  sem_b).wait()
            pltpu.async_copy(rows_b, acc.at[idx_d.at[j + 1]], sem_sb, add=True)
            pltpu.make_async_copy(rows_a, acc.at[idx_d.at[j]], sem_sa).wait()
            pltpu.async_copy(x_hbm.at[idx_s.at[j + 2]], rows_a, sem_a)
            pltpu.make_async_copy(rows_b, acc.at[idx_d.at[j + 1]],
                                  sem_sb).wait()
            pltpu.async_copy(x_hbm.at[idx_s.at[j + 3]], rows_b, sem_b)

        pltpu.make_async_copy(x_hbm.at[idx_s.at[CROWS]], rows_a, sem_a).wait()
        pltpu.make_async_copy(x_hbm.at[idx_s.at[CROWS + 1]], rows_b,
                              sem_b).wait()

    plsc.subcore_barrier()

    pltpu.sync_copy(acc.at[pl.ds(base, ROWS_PER_TILE)],
                    out_hbm.at[cid, pl.ds(base, ROWS_PER_TILE)])


def _sc_cnt(dst_hbm, zc_hbm, ones_hbm, cnt_hbm, idx_d, ones_v, cnt, sem):
    cid = lax.axis_index("c")
    sid = lax.axis_index("s")
    wid = sid * NC + cid

    base = sid * ROWS_PER_TILE
    pltpu.sync_copy(zc_hbm.at[pl.ds(base, ROWS_PER_TILE)],
                    cnt.at[pl.ds(base, ROWS_PER_TILE)])
    pltpu.sync_copy(ones_hbm, ones_v)
    plsc.subcore_barrier()

    erow0 = wid * EROWS_PER_W
    pltpu.sync_copy(dst_hbm.at[pl.ds(erow0, EROWS_PER_W)], idx_d)

    @pl.loop(0, EROWS_PER_W)
    def _(j):
        pltpu.sync_copy(ones_v, cnt.at[idx_d.at[j]], add=True)

    plsc.subcore_barrier()
    pltpu.sync_copy(cnt.at[pl.ds(base, ROWS_PER_TILE)],
                    cnt_hbm.at[cid, pl.ds(base, ROWS_PER_TILE)])


@functools.lru_cache(maxsize=None)
def _sc_agg_call():
    mesh = plsc.VectorSubcoreMesh(core_axis_name="c", subcore_axis_name="s",
                                  num_cores=NC, num_subcores=NS)
    return pl.kernel(
        _sc_agg,
        out_type=[jax.ShapeDtypeStruct((NC, N_PAD, D), jnp.float32)],
        mesh=mesh,
        scratch_types=[
            pltpu.VMEM((CROWS + 2, 128), jnp.int32),
            pltpu.VMEM((CROWS, 128), jnp.int32),
            pltpu.VMEM((128, D), jnp.float32),
            pltpu.VMEM((128, D), jnp.float32),
            pltpu.SemaphoreType.DMA,
            pltpu.SemaphoreType.DMA,
            pltpu.SemaphoreType.DMA,
            pltpu.SemaphoreType.DMA,
            pltpu.VMEM_SHARED((N_PAD, D), jnp.float32),
        ],
        name="sc_seg_agg")


@functools.lru_cache(maxsize=None)
def _sc_cnt_call():
    mesh = plsc.VectorSubcoreMesh(core_axis_name="c", subcore_axis_name="s",
                                  num_cores=NC, num_subcores=NS)
    return pl.kernel(
        _sc_cnt,
        out_type=[jax.ShapeDtypeStruct((NC, N_PAD, CW), jnp.float32)],
        mesh=mesh,
        scratch_types=[
            pltpu.VMEM((EROWS_PER_W, 128), jnp.int32),
            pltpu.VMEM((128, CW), jnp.float32),
            pltpu.VMEM_SHARED((N_PAD, CW), jnp.float32),
            pltpu.SemaphoreType.DMA,
        ],
        name="sc_seg_cnt")


BLK = 2000
NBLK = N // BLK


def _tc_layer(parts_ref, cnt_ref, x_ref, wl_ref, wr_ref, b_ref, o_ref):
    s = parts_ref[0] + parts_ref[1]
    deg = cnt_ref[0, :, 0:1] + cnt_ref[1, :, 0:1]
    mean = s / jnp.maximum(deg, 1.0)
    h = jnp.dot(mean, wl_ref[...], preferred_element_type=jnp.float32)
    h = h + jnp.dot(x_ref[...], wr_ref[...], preferred_element_type=jnp.float32)
    o_ref[...] = jnp.maximum(h + b_ref[...], 0.0)


def _tc_layer_call(parts, cnt, x, wl, wr, b):
    return pl.pallas_call(
        _tc_layer,
        grid=(NBLK,),
        in_specs=[
            pl.BlockSpec((NC, BLK, D), lambda i: (0, i, 0)),
            pl.BlockSpec((NC, BLK, CW), lambda i: (0, i, 0)),
            pl.BlockSpec((BLK, D), lambda i: (i, 0)),
            pl.BlockSpec((D, D), lambda i: (0, 0)),
            pl.BlockSpec((D, D), lambda i: (0, 0)),
            pl.BlockSpec((1, D), lambda i: (0, 0)),
        ],
        out_specs=pl.BlockSpec((BLK, D), lambda i: (i, 0)),
        out_shape=jax.ShapeDtypeStruct((N, D), jnp.float32),
    )(parts, cnt, x, wl, wr, b)


def _tc_final(parts_ref, cnt_ref, h1_ref, wl_ref, wr_ref, b_ref,
              batch_ref, w3_ref, b3_ref, w4_ref, b4_ref, o_ref,
              pool_acc, cntg_acc):
    i = pl.program_id(0)
    s = parts_ref[0] + parts_ref[1]
    deg = cnt_ref[0, :, 0:1] + cnt_ref[1, :, 0:1]
    mean = s / jnp.maximum(deg, 1.0)
    h = jnp.dot(mean, wl_ref[...], preferred_element_type=jnp.float32)
    h = h + jnp.dot(h1_ref[...], wr_ref[...], preferred_element_type=jnp.float32)
    h2 = jnp.maximum(h + b_ref[...], 0.0)

    giota = lax.broadcasted_iota(jnp.int32, (G, BLK), 0)
    onehot_t = (giota == batch_ref[0]).astype(jnp.float32)

    @pl.when(i == 0)
    def _():
        pool_acc[...] = jnp.zeros_like(pool_acc)
        cntg_acc[...] = jnp.zeros_like(cntg_acc)

    pool_acc[...] += jnp.dot(onehot_t, h2, preferred_element_type=jnp.float32)
    cntg_acc[...] += jnp.sum(onehot_t, axis=1, keepdims=True)

    @pl.when(i == NBLK - 1)
    def _():
        pooled = pool_acc[...] / jnp.maximum(cntg_acc[...], 1.0)
        z = jnp.dot(pooled, w3_ref[...], preferred_element_type=jnp.float32)
        z = jnp.maximum(z + b3_ref[...], 0.0)
        o_ref[...] = jnp.dot(z, w4_ref[...],
                             preferred_element_type=jnp.float32) + b4_ref[...]


def _tc_final_call(parts, cnt, h1, wl, wr, b, batch3, w3, b3, w4, b4):
    return pl.pallas_call(
        _tc_final,
        grid=(NBLK,),
        in_specs=[
            pl.BlockSpec((NC, BLK, D), lambda i: (0, i, 0)),
            pl.BlockSpec((NC, BLK, CW), lambda i: (0, i, 0)),
            pl.BlockSpec((BLK, D), lambda i: (i, 0)),
            pl.BlockSpec((D, D), lambda i: (0, 0)),
            pl.BlockSpec((D, D), lambda i: (0, 0)),
            pl.BlockSpec((1, D), lambda i: (0, 0)),
            pl.BlockSpec((1, 1, BLK), lambda i: (i, 0, 0)),
            pl.BlockSpec((D, D), lambda i: (0, 0)),
            pl.BlockSpec((1, D), lambda i: (0, 0)),
            pl.BlockSpec((D, D), lambda i: (0, 0)),
            pl.BlockSpec((1, D), lambda i: (0, 0)),
        ],
        out_specs=pl.BlockSpec((G, D), lambda i: (0, 0)),
        out_shape=jax.ShapeDtypeStruct((G, D), jnp.float32),
        scratch_shapes=[
            pltpu.VMEM((G, D), jnp.float32),
            pltpu.VMEM((G, 1), jnp.float32),
        ],
    )(parts, cnt, h1, wl, wr, b, batch3, w3, b3, w4, b4)


def kernel(x, edge_index, batch, W1l, b1l, W1r, W2l, b2l, W2r, W3, b3, W4, b4):
    src = edge_index[0]
    dst = edge_index[1]
    npad = E_PAD - E
    src2 = jnp.concatenate(
        [src, jnp.zeros((npad,), jnp.int32)]).reshape(EROWS, 128)
    dst2 = jnp.concatenate(
        [dst, jnp.full((npad,), PAD_DST, jnp.int32)]).reshape(EROWS, 128)
    z2d = jnp.zeros((N_PAD, D), jnp.float32)
    ones = jnp.ones((128, CW), jnp.float32)

    (cnt,) = _sc_cnt_call()(dst2, z2d, ones)
    (parts1,) = _sc_agg_call()(x, src2, dst2, z2d)
    h1 = _tc_layer_call(parts1, cnt, x, W1l, W1r, b1l.reshape(1, D))
    (parts2,) = _sc_agg_call()(h1, src2, dst2, z2d)
    batch3 = batch.reshape(NBLK, 1, BLK)
    return _tc_final_call(parts2, cnt, h1, W2l, W2r, b2l.reshape(1, D),
                          batch3, W3, b3.reshape(1, D), W4, b4.reshape(1, D))
